# baseline (speedup 1.0000x reference)
import jax
import jax.numpy as jnp
from jax import lax
from jax.experimental import pallas as pl
from jax.experimental.pallas import tpu as pltpu

_TOKENS = 8192
_HIDDEN = 2048
_EXPERTS = 64
_K = 8
_BT = 512


def _router_body(x_ref, wt_ref, b_ref, scores_ref, idx_ref):
    x = x_ref[...]
    logits = jnp.dot(x, wt_ref[...], preferred_element_type=jnp.float32)
    logits = logits + b_ref[...]

    neg_inf = jnp.float32(-jnp.inf)
    work = logits
    m_cols = []
    for _ in range(_K):
        m = jnp.max(work, axis=1, keepdims=True)
        work = jnp.where(work == m, neg_inf, work)
        m_cols.append(m)

    sel_mask = work == neg_inf
    e = jnp.where(sel_mask, jnp.exp(logits - m_cols[0]), 0.0)
    s = jnp.sum(e, axis=1, keepdims=True)
    scores_ref[...] = e / s

    rank = jnp.zeros(logits.shape, dtype=jnp.int32)
    for k in range(1, _K):
        rank = rank + jnp.where(logits < m_cols[k - 1], 1, 0)

    iota = lax.broadcasted_iota(jnp.int32, logits.shape, 1)
    sub = jnp.where(rank >= 4, rank - 4, rank)
    shift = 6 * (3 - sub)
    contrib = jnp.where(sel_mask, iota << shift, 0)
    lo_mask = rank < 4
    w1 = jnp.sum(jnp.where(lo_mask, contrib, 0).astype(jnp.float32),
                 axis=1, keepdims=True)
    w2 = jnp.sum(jnp.where(jnp.logical_and(sel_mask, ~lo_mask), contrib, 0)
                 .astype(jnp.float32), axis=1, keepdims=True)

    w = jnp.concatenate([w1] * 4 + [w2] * 4, axis=1).astype(jnp.int32)
    kio = lax.broadcasted_iota(jnp.int32, (logits.shape[0], _K), 1)
    ksub = jnp.where(kio >= 4, kio - 4, kio)
    idx_ref[...] = (w >> (6 * (3 - ksub))) & (_EXPERTS - 1)


@jax.jit
def kernel(hidden_states, weight, bias):
    x = hidden_states.reshape(-1, _HIDDEN)
    wt = weight.T
    b = bias.reshape(1, _EXPERTS)
    grid = (_TOKENS // _BT,)
    scores, idx = pl.pallas_call(
        _router_body,
        grid=grid,
        in_specs=[
            pl.BlockSpec((_BT, _HIDDEN), lambda i: (i, 0)),
            pl.BlockSpec((_HIDDEN, _EXPERTS), lambda i: (0, 0)),
            pl.BlockSpec((1, _EXPERTS), lambda i: (0, 0)),
        ],
        out_specs=[
            pl.BlockSpec((_BT, _EXPERTS), lambda i: (i, 0)),
            pl.BlockSpec((_BT, _K), lambda i: (i, 0)),
        ],
        out_shape=[
            jax.ShapeDtypeStruct((_TOKENS, _EXPERTS), jnp.float32),
            jax.ShapeDtypeStruct((_TOKENS, _K), jnp.int32),
        ],
        compiler_params=pltpu.CompilerParams(
            dimension_semantics=("arbitrary",),
        ),
    )(x, wt, b)
    return (scores, idx)


# speedup vs baseline: 1.1244x; 1.1244x over previous
import jax
import jax.numpy as jnp
from jax import lax
from jax.experimental import pallas as pl
from jax.experimental.pallas import tpu as pltpu

_TOKENS = 8192
_HIDDEN = 2048
_EXPERTS = 64
_K = 8
_BT = 1024


def _router_body(x_ref, wt_ref, b_ref, scores_ref, idx_ref):
    x = x_ref[...]
    logits = jnp.dot(x, wt_ref[...], preferred_element_type=jnp.float32)
    logits = logits + b_ref[...]

    neg_inf = jnp.float32(-jnp.inf)
    work = logits
    m_cols = []
    for _ in range(_K):
        m = jnp.max(work, axis=1, keepdims=True)
        work = jnp.where(work == m, neg_inf, work)
        m_cols.append(m)

    sel_mask = work == neg_inf
    e = jnp.where(sel_mask, jnp.exp(logits - m_cols[0]), 0.0)
    s = jnp.sum(e, axis=1, keepdims=True)
    scores_ref[...] = e / s

    b2 = logits < m_cols[3]
    b1 = logits < jnp.where(b2, m_cols[5], m_cols[1])
    p2a = jnp.where(b1, m_cols[2], m_cols[0])
    p2b = jnp.where(b1, m_cols[6], m_cols[4])
    b0 = logits < jnp.where(b2, p2b, p2a)

    iota = lax.broadcasted_iota(jnp.int32, logits.shape, 1)
    sub = jnp.where(b1, 2, 0) + jnp.where(b0, 1, 0)
    contrib = jnp.where(sel_mask, iota << (18 - 6 * sub), 0)
    c1 = jnp.where(b2, 0, contrib)
    w1 = jnp.sum(c1.astype(jnp.float32), axis=1, keepdims=True)
    w2 = jnp.sum((contrib - c1).astype(jnp.float32), axis=1, keepdims=True)

    w = jnp.concatenate([w1] * 4 + [w2] * 4, axis=1).astype(jnp.int32)
    kio = lax.broadcasted_iota(jnp.int32, (logits.shape[0], _K), 1)
    ksub = jnp.where(kio >= 4, kio - 4, kio)
    idx_ref[...] = (w >> (6 * (3 - ksub))) & (_EXPERTS - 1)


@jax.jit
def kernel(hidden_states, weight, bias):
    x = hidden_states.reshape(-1, _HIDDEN)
    wt = weight.T
    b = bias.reshape(1, _EXPERTS)
    grid = (_TOKENS // _BT,)
    scores, idx = pl.pallas_call(
        _router_body,
        grid=grid,
        in_specs=[
            pl.BlockSpec((_BT, _HIDDEN), lambda i: (i, 0)),
            pl.BlockSpec((_HIDDEN, _EXPERTS), lambda i: (0, 0)),
            pl.BlockSpec((1, _EXPERTS), lambda i: (0, 0)),
        ],
        out_specs=[
            pl.BlockSpec((_BT, _EXPERTS), lambda i: (i, 0)),
            pl.BlockSpec((_BT, _K), lambda i: (i, 0)),
        ],
        out_shape=[
            jax.ShapeDtypeStruct((_TOKENS, _EXPERTS), jnp.float32),
            jax.ShapeDtypeStruct((_TOKENS, _K), jnp.int32),
        ],
        compiler_params=pltpu.CompilerParams(
            dimension_semantics=("arbitrary",),
        ),
    )(x, wt, b)
    return (scores, idx)
